# Initial kernel scaffold; baseline (speedup 1.0000x reference)
import jax
import jax.numpy as jnp
from jax import lax
from jax.experimental import pallas as pl
from jax.experimental.pallas import tpu as pltpu
from jax.experimental.pallas import tpu_sc as plsc

_N_NODES = 10000
_HID = 16
_NC = 2
_NS = 16
_NW = _NC * _NS
_CHUNK = 128
_ROWS_PER_TILE = _N_NODES // _NS


def _propagate_body(n_chunks):
    def body(sup_hbm, src_hbm, dst_hbm, w_hbm, out_hbm,
             src_v, dst_v, w_v, rows_v, zero_v, acc_sh):
        c = lax.axis_index("core")
        s = lax.axis_index("subcore")
        wid = c * _NS + s

        pltpu.sync_copy(src_hbm.at[wid], src_v)
        pltpu.sync_copy(dst_hbm.at[wid], dst_v)
        pltpu.sync_copy(w_hbm.at[wid], w_v)

        @pl.loop(0, _ROWS_PER_TILE)
        def _zero(i):
            zero_v[i, :] = jnp.zeros((_HID,), jnp.float32)

        pltpu.sync_copy(
            zero_v, acc_sh.at[pl.ds(s * _ROWS_PER_TILE, _ROWS_PER_TILE)])
        plsc.subcore_barrier()

        @pl.loop(0, n_chunks)
        def _chunk(ch):
            pltpu.sync_copy(sup_hbm.at[src_v.at[ch]], rows_v)

            @pl.loop(0, _CHUNK)
            def _edge(i):
                wv = plsc.load_gather(
                    w_v,
                    [jnp.full((_HID,), ch, jnp.int32),
                     jnp.full((_HID,), i, jnp.int32)])
                rows_v[i, :] = rows_v[i, :] * wv

            pltpu.sync_copy(rows_v, acc_sh.at[dst_v.at[ch]], add=True)

        plsc.subcore_barrier()
        rs = pl.ds(s * _ROWS_PER_TILE, _ROWS_PER_TILE)
        pltpu.sync_copy(acc_sh.at[rs], out_hbm.at[c].at[rs])

    return body


def _propagate(support, src3, dst3, w3):
    n_chunks = src3.shape[1]
    kfn = pl.kernel(
        _propagate_body(n_chunks),
        out_type=jax.ShapeDtypeStruct((_NC, _N_NODES, _HID), jnp.float32),
        mesh=plsc.VectorSubcoreMesh(
            core_axis_name="core", subcore_axis_name="subcore"),
        scratch_types=[
            pltpu.VMEM((n_chunks, _CHUNK), jnp.int32),
            pltpu.VMEM((n_chunks, _CHUNK), jnp.int32),
            pltpu.VMEM((n_chunks, _CHUNK), jnp.float32),
            pltpu.VMEM((_CHUNK, _HID), jnp.float32),
            pltpu.VMEM((_ROWS_PER_TILE, _HID), jnp.float32),
            pltpu.VMEM_SHARED((_N_NODES, _HID), jnp.float32),
        ],
    )
    return kfn(support, src3, dst3, w3)


def _mm1_body(x_ref, w_ref, o_ref):
    o_ref[...] = jnp.dot(x_ref[...], w_ref[...],
                         preferred_element_type=jnp.float32)


def _mm2_body(p_ref, b_ref, w_ref, o_ref):
    h = p_ref[0] + p_ref[1] + b_ref[...]
    h = jnp.maximum(h, 0.0)
    o_ref[...] = jnp.dot(h, w_ref[...], preferred_element_type=jnp.float32)


def _fin_body(q_ref, b_ref, o_ref):
    o_ref[...] = q_ref[0] + q_ref[1] + b_ref[...]


def kernel(feature, edge_index, edge_weight, W1, b1, W2, b2):
    n_edges = edge_index.shape[1]
    n_chunks = -(-n_edges // (_NW * _CHUNK))
    e_pad = _NW * n_chunks * _CHUNK

    src = edge_index[0].astype(jnp.int32)
    dst = edge_index[1].astype(jnp.int32)
    w = edge_weight.astype(jnp.float32)
    pad = e_pad - n_edges
    src3 = jnp.concatenate([src, jnp.zeros((pad,), jnp.int32)]
                           ).reshape(_NW, n_chunks, _CHUNK)
    dst3 = jnp.concatenate([dst, jnp.zeros((pad,), jnp.int32)]
                           ).reshape(_NW, n_chunks, _CHUNK)
    w3 = jnp.concatenate([w, jnp.zeros((pad,), jnp.float32)]
                         ).reshape(_NW, n_chunks, _CHUNK)

    n = feature.shape[0]
    support1 = pl.pallas_call(
        _mm1_body,
        out_shape=jax.ShapeDtypeStruct((n, _HID), jnp.float32),
    )(feature, W1)

    p = _propagate(support1, src3, dst3, w3)

    W2p = jnp.zeros((_HID, _HID), jnp.float32).at[:, :W2.shape[1]].set(W2)
    support2 = pl.pallas_call(
        _mm2_body,
        out_shape=jax.ShapeDtypeStruct((n, _HID), jnp.float32),
    )(p, b1.reshape(1, _HID), W2p)

    q = _propagate(support2, src3, dst3, w3)

    b2p = jnp.zeros((1, _HID), jnp.float32).at[0, :b2.shape[0]].set(b2)
    out16 = pl.pallas_call(
        _fin_body,
        out_shape=jax.ShapeDtypeStruct((n, _HID), jnp.float32),
    )(q, b2p)
    return out16[:, :b2.shape[0]]


# speedup vs baseline: 6.4030x; 6.4030x over previous
import dataclasses

import jax
import jax.numpy as jnp
from jax import lax
from jax.experimental import pallas as pl
from jax.experimental.pallas import tpu as pltpu
from jax.experimental.pallas import tpu_sc as plsc

_N_NODES = 10000
_HID = 16
_NC = 2
_NS = 16
_NW = _NC * _NS
_CHUNK = 128
_N_PAD = 10240
_ROWS_PER_TILE = _N_PAD // _NS


def _propagate_body(n_chunks):
    def body(sup_hbm, src_hbm, dst_hbm, w_hbm, out_hbm,
             src_v, dst_v, w_v, rows_v, zero_v, acc_sh):
        c = lax.axis_index("core")
        s = lax.axis_index("subcore")
        wid = c * _NS + s

        pltpu.sync_copy(src_hbm.at[wid], src_v)
        pltpu.sync_copy(dst_hbm.at[wid], dst_v)
        pltpu.sync_copy(w_hbm.at[wid], w_v)

        @pl.loop(0, _ROWS_PER_TILE)
        def _zero(i):
            zero_v[i, :] = jnp.zeros((_HID,), jnp.float32)

        pltpu.sync_copy(
            zero_v, acc_sh.at[pl.ds(s * _ROWS_PER_TILE, _ROWS_PER_TILE)])
        plsc.subcore_barrier()

        @pl.loop(0, n_chunks)
        def _chunk(ch):
            pltpu.sync_copy(sup_hbm.at[src_v.at[ch]], rows_v)

            @pl.loop(0, _CHUNK)
            def _edge(i):
                wv = plsc.load_gather(
                    w_v,
                    [jnp.full((_HID,), ch, jnp.int32),
                     jnp.full((_HID,), i, jnp.int32)])
                rows_v[i, :] = rows_v[i, :] * wv

            pltpu.sync_copy(rows_v, acc_sh.at[dst_v.at[ch]], add=True)

        plsc.subcore_barrier()
        rs = pl.ds(s * _ROWS_PER_TILE, _ROWS_PER_TILE)
        pltpu.sync_copy(acc_sh.at[rs], out_hbm.at[c].at[rs])

    return body


def _sc_compiler_params():
    cp = pltpu.CompilerParams()
    fields = pltpu.CompilerParams.__dataclass_fields__
    if "needs_layout_passes" in fields:
        cp = dataclasses.replace(cp, needs_layout_passes=False)
    if "use_tc_tiling_on_sc" in fields:
        cp = dataclasses.replace(cp, use_tc_tiling_on_sc=False)
    return cp


def _propagate(support, src3, dst3, w3):
    n_chunks = src3.shape[1]
    kfn = pl.kernel(
        _propagate_body(n_chunks),
        out_type=jax.ShapeDtypeStruct((_NC, _N_PAD, _HID), jnp.float32),
        mesh=plsc.VectorSubcoreMesh(
            core_axis_name="core", subcore_axis_name="subcore"),
        scratch_types=[
            pltpu.VMEM((n_chunks, _CHUNK), jnp.int32),
            pltpu.VMEM((n_chunks, _CHUNK), jnp.int32),
            pltpu.VMEM((n_chunks, _CHUNK), jnp.float32),
            pltpu.VMEM((_CHUNK, _HID), jnp.float32),
            pltpu.VMEM((_ROWS_PER_TILE, _HID), jnp.float32),
            pltpu.VMEM_SHARED((_N_PAD, _HID), jnp.float32),
        ],
        compiler_params=_sc_compiler_params(),
    )
    return kfn(support, src3, dst3, w3)[:, :_N_NODES]


def _mm1_body(x_ref, w_ref, o_ref):
    o_ref[...] = jnp.dot(x_ref[...], w_ref[...],
                         preferred_element_type=jnp.float32)


def _mm2_body(p_ref, b_ref, w_ref, o_ref):
    h = p_ref[0] + p_ref[1] + b_ref[...]
    h = jnp.maximum(h, 0.0)
    o_ref[...] = jnp.dot(h, w_ref[...], preferred_element_type=jnp.float32)


def _fin_body(q_ref, b_ref, o_ref):
    o_ref[...] = q_ref[0] + q_ref[1] + b_ref[...]


def kernel(feature, edge_index, edge_weight, W1, b1, W2, b2):
    n_edges = edge_index.shape[1]
    n_chunks = -(-n_edges // (_NW * _CHUNK))
    e_pad = _NW * n_chunks * _CHUNK

    src = edge_index[0].astype(jnp.int32)
    dst = edge_index[1].astype(jnp.int32)
    w = edge_weight.astype(jnp.float32)
    pad = e_pad - n_edges
    src3 = jnp.concatenate([src, jnp.zeros((pad,), jnp.int32)]
                           ).reshape(_NW, n_chunks, _CHUNK)
    dst3 = jnp.concatenate([dst, jnp.zeros((pad,), jnp.int32)]
                           ).reshape(_NW, n_chunks, _CHUNK)
    w3 = jnp.concatenate([w, jnp.zeros((pad,), jnp.float32)]
                         ).reshape(_NW, n_chunks, _CHUNK)

    n = feature.shape[0]
    support1 = pl.pallas_call(
        _mm1_body,
        out_shape=jax.ShapeDtypeStruct((n, _HID), jnp.float32),
    )(feature, W1)

    p = _propagate(support1, src3, dst3, w3)

    W2p = jnp.zeros((_HID, _HID), jnp.float32).at[:, :W2.shape[1]].set(W2)
    support2 = pl.pallas_call(
        _mm2_body,
        out_shape=jax.ShapeDtypeStruct((n, _HID), jnp.float32),
    )(p, b1.reshape(1, _HID), W2p)

    q = _propagate(support2, src3, dst3, w3)

    b2p = jnp.zeros((1, _HID), jnp.float32).at[0, :b2.shape[0]].set(b2)
    out16 = pl.pallas_call(
        _fin_body,
        out_shape=jax.ShapeDtypeStruct((n, _HID), jnp.float32),
    )(q, b2p)
    return out16[:, :b2.shape[0]]


# speedup vs baseline: 7.8140x; 1.2204x over previous
import dataclasses

import jax
import jax.numpy as jnp
from jax import lax
from jax.experimental import pallas as pl
from jax.experimental.pallas import tpu as pltpu
from jax.experimental.pallas import tpu_sc as plsc

_N_NODES = 10000
_HID = 16
_NC = 2
_NS = 16
_NW = _NC * _NS
_CHUNK = 128
_GATHER_DNUMS = lax.GatherDimensionNumbers(
    offset_dims=(), collapsed_slice_dims=(0,), start_index_map=(0,))


def _lane_broadcast(vec, lane):
    idx = jnp.full((_HID, 1), lane, jnp.int32)
    return lax.gather(vec, idx, _GATHER_DNUMS, slice_sizes=(1,),
                      mode=lax.GatherScatterMode.PROMISE_IN_BOUNDS)

_N_PAD = 10240
_ROWS_PER_TILE = _N_PAD // _NS


def _propagate_body(n_chunks):
    def body(sup_hbm, src_hbm, dst_hbm, w_hbm, out_hbm,
             src_v, dst_v, w_v, rows_v, zero_v, acc_sh):
        c = lax.axis_index("core")
        s = lax.axis_index("subcore")
        wid = c * _NS + s

        pltpu.sync_copy(src_hbm.at[wid], src_v)
        pltpu.sync_copy(dst_hbm.at[wid], dst_v)
        pltpu.sync_copy(w_hbm.at[wid], w_v)

        @pl.loop(0, _ROWS_PER_TILE)
        def _zero(i):
            zero_v[i, :] = jnp.zeros((_HID,), jnp.float32)

        pltpu.sync_copy(
            zero_v, acc_sh.at[pl.ds(s * _ROWS_PER_TILE, _ROWS_PER_TILE)])
        plsc.subcore_barrier()

        @pl.loop(0, n_chunks)
        def _chunk(ch):
            pltpu.sync_copy(sup_hbm.at[src_v.at[ch]], rows_v)

            @pl.loop(0, _CHUNK // _HID, unroll=2)
            def _grp(g):
                base = g * _HID
                wv16 = w_v[ch, pl.ds(base, _HID)]
                for j in range(_HID):
                    wj = _lane_broadcast(wv16, j)
                    rows_v[base + j, :] = rows_v[base + j, :] * wj

            pltpu.sync_copy(rows_v, acc_sh.at[dst_v.at[ch]], add=True)

        plsc.subcore_barrier()
        rs = pl.ds(s * _ROWS_PER_TILE, _ROWS_PER_TILE)
        pltpu.sync_copy(acc_sh.at[rs], out_hbm.at[c].at[rs])

    return body


def _sc_compiler_params():
    cp = pltpu.CompilerParams()
    fields = pltpu.CompilerParams.__dataclass_fields__
    if "needs_layout_passes" in fields:
        cp = dataclasses.replace(cp, needs_layout_passes=False)
    if "use_tc_tiling_on_sc" in fields:
        cp = dataclasses.replace(cp, use_tc_tiling_on_sc=False)
    return cp


def _propagate(support, src3, dst3, w3):
    n_chunks = src3.shape[1]
    kfn = pl.kernel(
        _propagate_body(n_chunks),
        out_type=jax.ShapeDtypeStruct((_NC, _N_PAD, _HID), jnp.float32),
        mesh=plsc.VectorSubcoreMesh(
            core_axis_name="core", subcore_axis_name="subcore"),
        scratch_types=[
            pltpu.VMEM((n_chunks, _CHUNK), jnp.int32),
            pltpu.VMEM((n_chunks, _CHUNK), jnp.int32),
            pltpu.VMEM((n_chunks, _CHUNK), jnp.float32),
            pltpu.VMEM((_CHUNK, _HID), jnp.float32),
            pltpu.VMEM((_ROWS_PER_TILE, _HID), jnp.float32),
            pltpu.VMEM_SHARED((_N_PAD, _HID), jnp.float32),
        ],
        compiler_params=_sc_compiler_params(),
    )
    return kfn(support, src3, dst3, w3)[:, :_N_NODES]


def _mm1_body(x_ref, w_ref, o_ref):
    o_ref[...] = jnp.dot(x_ref[...], w_ref[...],
                         preferred_element_type=jnp.float32)


def _mm2_body(p_ref, b_ref, w_ref, o_ref):
    h = p_ref[0] + p_ref[1] + b_ref[...]
    h = jnp.maximum(h, 0.0)
    o_ref[...] = jnp.dot(h, w_ref[...], preferred_element_type=jnp.float32)


def _fin_body(q_ref, b_ref, o_ref):
    o_ref[...] = q_ref[0] + q_ref[1] + b_ref[...]


def kernel(feature, edge_index, edge_weight, W1, b1, W2, b2):
    n_edges = edge_index.shape[1]
    n_chunks = -(-n_edges // (_NW * _CHUNK))
    e_pad = _NW * n_chunks * _CHUNK

    src = edge_index[0].astype(jnp.int32)
    dst = edge_index[1].astype(jnp.int32)
    w = edge_weight.astype(jnp.float32)
    pad = e_pad - n_edges
    src3 = jnp.concatenate([src, jnp.zeros((pad,), jnp.int32)]
                           ).reshape(_NW, n_chunks, _CHUNK)
    dst3 = jnp.concatenate([dst, jnp.zeros((pad,), jnp.int32)]
                           ).reshape(_NW, n_chunks, _CHUNK)
    w3 = jnp.concatenate([w, jnp.zeros((pad,), jnp.float32)]
                         ).reshape(_NW, n_chunks, _CHUNK)

    n = feature.shape[0]
    support1 = pl.pallas_call(
        _mm1_body,
        out_shape=jax.ShapeDtypeStruct((n, _HID), jnp.float32),
    )(feature, W1)

    p = _propagate(support1, src3, dst3, w3)

    W2p = jnp.zeros((_HID, _HID), jnp.float32).at[:, :W2.shape[1]].set(W2)
    support2 = pl.pallas_call(
        _mm2_body,
        out_shape=jax.ShapeDtypeStruct((n, _HID), jnp.float32),
    )(p, b1.reshape(1, _HID), W2p)

    q = _propagate(support2, src3, dst3, w3)

    b2p = jnp.zeros((1, _HID), jnp.float32).at[0, :b2.shape[0]].set(b2)
    out16 = pl.pallas_call(
        _fin_body,
        out_shape=jax.ShapeDtypeStruct((n, _HID), jnp.float32),
    )(q, b2p)
    return out16[:, :b2.shape[0]]


# speedup vs baseline: 9.2616x; 1.1853x over previous
import dataclasses

import jax
import jax.numpy as jnp
from jax import lax
from jax.experimental import pallas as pl
from jax.experimental.pallas import tpu as pltpu
from jax.experimental.pallas import tpu_sc as plsc

_N_NODES = 10000
_HID = 16
_NC = 2
_NS = 16
_NW = _NC * _NS
_CHUNK = 128
_GATHER_DNUMS = lax.GatherDimensionNumbers(
    offset_dims=(), collapsed_slice_dims=(0,), start_index_map=(0,))


def _lane_broadcast(vec, lane):
    idx = jnp.full((_HID, 1), lane, jnp.int32)
    return lax.gather(vec, idx, _GATHER_DNUMS, slice_sizes=(1,),
                      mode=lax.GatherScatterMode.PROMISE_IN_BOUNDS)

_N_PAD = 10240
_ROWS_PER_TILE = _N_PAD // _NS


def _propagate_body(n_chunks):
    assert n_chunks % 2 == 0 and n_chunks >= 4

    def body(sup_hbm, src_hbm, dst_hbm, w_hbm, out_hbm,
             src_v, dst_v, w_v, rows0, rows1, zero_v, acc_sh,
             gsem0, gsem1, ssem0, ssem1):
        c = lax.axis_index("core")
        s = lax.axis_index("subcore")
        wid = c * _NS + s

        pltpu.sync_copy(src_hbm.at[wid], src_v)
        pltpu.sync_copy(dst_hbm.at[wid], dst_v)
        pltpu.sync_copy(w_hbm.at[wid], w_v)

        pltpu.async_copy(sup_hbm.at[src_v.at[0]], rows0, gsem0)
        pltpu.async_copy(sup_hbm.at[src_v.at[1]], rows1, gsem1)

        @pl.loop(0, _ROWS_PER_TILE)
        def _zero(i):
            zero_v[i, :] = jnp.zeros((_HID,), jnp.float32)

        pltpu.sync_copy(
            zero_v, acc_sh.at[pl.ds(s * _ROWS_PER_TILE, _ROWS_PER_TILE)])
        plsc.subcore_barrier()

        def scale(rows, ch):
            @pl.loop(0, _CHUNK // _HID, unroll=2)
            def _grp(g):
                base = g * _HID
                wv16 = w_v[ch, pl.ds(base, _HID)]
                for j in range(_HID):
                    wj = _lane_broadcast(wv16, j)
                    rows[base + j, :] = rows[base + j, :] * wj

        @pl.loop(0, n_chunks, step=2)
        def _chunk(ch):
            pltpu.make_async_copy(
                sup_hbm.at[src_v.at[ch]], rows0, gsem0).wait()
            scale(rows0, ch)
            s0 = pltpu.async_copy(
                rows0, acc_sh.at[dst_v.at[ch]], ssem0, add=True)

            pltpu.make_async_copy(
                sup_hbm.at[src_v.at[ch + 1]], rows1, gsem1).wait()
            scale(rows1, ch + 1)
            s1 = pltpu.async_copy(
                rows1, acc_sh.at[dst_v.at[ch + 1]], ssem1, add=True)

            s0.wait()

            @pl.when(ch + 2 < n_chunks)
            def _():
                pltpu.async_copy(sup_hbm.at[src_v.at[ch + 2]], rows0, gsem0)

            s1.wait()

            @pl.when(ch + 3 < n_chunks)
            def _():
                pltpu.async_copy(sup_hbm.at[src_v.at[ch + 3]], rows1, gsem1)

        plsc.subcore_barrier()
        rs = pl.ds(s * _ROWS_PER_TILE, _ROWS_PER_TILE)
        pltpu.sync_copy(acc_sh.at[rs], out_hbm.at[c].at[rs])

    return body


def _sc_compiler_params():
    cp = pltpu.CompilerParams()
    fields = pltpu.CompilerParams.__dataclass_fields__
    if "needs_layout_passes" in fields:
        cp = dataclasses.replace(cp, needs_layout_passes=False)
    if "use_tc_tiling_on_sc" in fields:
        cp = dataclasses.replace(cp, use_tc_tiling_on_sc=False)
    return cp


def _propagate(support, src3, dst3, w3):
    n_chunks = src3.shape[1]
    kfn = pl.kernel(
        _propagate_body(n_chunks),
        out_type=jax.ShapeDtypeStruct((_NC, _N_PAD, _HID), jnp.float32),
        mesh=plsc.VectorSubcoreMesh(
            core_axis_name="core", subcore_axis_name="subcore"),
        scratch_types=[
            pltpu.VMEM((n_chunks, _CHUNK), jnp.int32),
            pltpu.VMEM((n_chunks, _CHUNK), jnp.int32),
            pltpu.VMEM((n_chunks, _CHUNK), jnp.float32),
            pltpu.VMEM((_CHUNK, _HID), jnp.float32),
            pltpu.VMEM((_CHUNK, _HID), jnp.float32),
            pltpu.VMEM((_ROWS_PER_TILE, _HID), jnp.float32),
            pltpu.VMEM_SHARED((_N_PAD, _HID), jnp.float32),
            pltpu.SemaphoreType.DMA,
            pltpu.SemaphoreType.DMA,
            pltpu.SemaphoreType.DMA,
            pltpu.SemaphoreType.DMA,
        ],
        compiler_params=_sc_compiler_params(),
    )
    return kfn(support, src3, dst3, w3)[:, :_N_NODES]


def _mm1_body(x_ref, w_ref, o_ref):
    o_ref[...] = jnp.dot(x_ref[...], w_ref[...],
                         preferred_element_type=jnp.float32)


def _mm2_body(p_ref, b_ref, w_ref, o_ref):
    h = p_ref[0] + p_ref[1] + b_ref[...]
    h = jnp.maximum(h, 0.0)
    o_ref[...] = jnp.dot(h, w_ref[...], preferred_element_type=jnp.float32)


def _fin_body(q_ref, b_ref, o_ref):
    o_ref[...] = q_ref[0] + q_ref[1] + b_ref[...]


def kernel(feature, edge_index, edge_weight, W1, b1, W2, b2):
    n_edges = edge_index.shape[1]
    n_chunks = -(-n_edges // (_NW * _CHUNK))
    e_pad = _NW * n_chunks * _CHUNK

    src = edge_index[0].astype(jnp.int32)
    dst = edge_index[1].astype(jnp.int32)
    w = edge_weight.astype(jnp.float32)
    pad = e_pad - n_edges
    src3 = jnp.concatenate([src, jnp.zeros((pad,), jnp.int32)]
                           ).reshape(_NW, n_chunks, _CHUNK)
    dst3 = jnp.concatenate([dst, jnp.zeros((pad,), jnp.int32)]
                           ).reshape(_NW, n_chunks, _CHUNK)
    w3 = jnp.concatenate([w, jnp.zeros((pad,), jnp.float32)]
                         ).reshape(_NW, n_chunks, _CHUNK)

    n = feature.shape[0]
    support1 = pl.pallas_call(
        _mm1_body,
        out_shape=jax.ShapeDtypeStruct((n, _HID), jnp.float32),
    )(feature, W1)

    p = _propagate(support1, src3, dst3, w3)

    W2p = jnp.zeros((_HID, _HID), jnp.float32).at[:, :W2.shape[1]].set(W2)
    support2 = pl.pallas_call(
        _mm2_body,
        out_shape=jax.ShapeDtypeStruct((n, _HID), jnp.float32),
    )(p, b1.reshape(1, _HID), W2p)

    q = _propagate(support2, src3, dst3, w3)

    b2p = jnp.zeros((1, _HID), jnp.float32).at[0, :b2.shape[0]].set(b2)
    out16 = pl.pallas_call(
        _fin_body,
        out_shape=jax.ShapeDtypeStruct((n, _HID), jnp.float32),
    )(q, b2p)
    return out16[:, :b2.shape[0]]


# speedup vs baseline: 9.3340x; 1.0078x over previous
import dataclasses

import jax
import jax.numpy as jnp
from jax import lax
from jax.experimental import pallas as pl
from jax.experimental.pallas import tpu as pltpu
from jax.experimental.pallas import tpu_sc as plsc

_N_NODES = 10000
_HID = 16
_NC = 2
_NS = 16
_NW = _NC * _NS
_CHUNK = 128
_GATHER_DNUMS = lax.GatherDimensionNumbers(
    offset_dims=(), collapsed_slice_dims=(0,), start_index_map=(0,))


def _lane_broadcast(vec, lane):
    idx = jnp.full((_HID, 1), lane, jnp.int32)
    return lax.gather(vec, idx, _GATHER_DNUMS, slice_sizes=(1,),
                      mode=lax.GatherScatterMode.PROMISE_IN_BOUNDS)

_N_PAD = 10240
_ROWS_PER_TILE = _N_PAD // _NS


def _propagate_body(n_chunks):
    assert n_chunks % 2 == 0 and n_chunks >= 4

    def body(sup_hbm, src_hbm, dst_hbm, w_hbm, out_hbm,
             src_v, dst_v, w_v, rows0, rows1, zero_v, acc_sh,
             gsem0, gsem1, ssem0, ssem1):
        c = lax.axis_index("core")
        s = lax.axis_index("subcore")
        wid = c * _NS + s

        c_src = pltpu.async_copy(src_hbm.at[wid], src_v, gsem0)
        c_dst = pltpu.async_copy(dst_hbm.at[wid], dst_v, gsem1)
        c_w = pltpu.async_copy(w_hbm.at[wid], w_v, ssem0)
        c_src.wait()
        c_dst.wait()
        c_w.wait()

        pltpu.async_copy(sup_hbm.at[src_v.at[0]], rows0, gsem0)
        pltpu.async_copy(sup_hbm.at[src_v.at[1]], rows1, gsem1)

        @pl.loop(0, _ROWS_PER_TILE)
        def _zero(i):
            zero_v[i, :] = jnp.zeros((_HID,), jnp.float32)

        pltpu.sync_copy(
            zero_v, acc_sh.at[pl.ds(s * _ROWS_PER_TILE, _ROWS_PER_TILE)])
        plsc.subcore_barrier()

        def scale(rows, ch):
            @pl.loop(0, _CHUNK // _HID, unroll=4)
            def _grp(g):
                base = g * _HID
                wv16 = w_v[ch, pl.ds(base, _HID)]
                for j in range(_HID):
                    wj = _lane_broadcast(wv16, j)
                    rows[base + j, :] = rows[base + j, :] * wj

        @pl.loop(0, n_chunks, step=2)
        def _chunk(ch):
            pltpu.make_async_copy(
                sup_hbm.at[src_v.at[ch]], rows0, gsem0).wait()
            scale(rows0, ch)
            s0 = pltpu.async_copy(
                rows0, acc_sh.at[dst_v.at[ch]], ssem0, add=True)

            pltpu.make_async_copy(
                sup_hbm.at[src_v.at[ch + 1]], rows1, gsem1).wait()
            scale(rows1, ch + 1)
            s1 = pltpu.async_copy(
                rows1, acc_sh.at[dst_v.at[ch + 1]], ssem1, add=True)

            s0.wait()

            @pl.when(ch + 2 < n_chunks)
            def _():
                pltpu.async_copy(sup_hbm.at[src_v.at[ch + 2]], rows0, gsem0)

            s1.wait()

            @pl.when(ch + 3 < n_chunks)
            def _():
                pltpu.async_copy(sup_hbm.at[src_v.at[ch + 3]], rows1, gsem1)

        plsc.subcore_barrier()
        rs = pl.ds(s * _ROWS_PER_TILE, _ROWS_PER_TILE)
        pltpu.sync_copy(acc_sh.at[rs], out_hbm.at[c].at[rs])

    return body


def _sc_compiler_params():
    cp = pltpu.CompilerParams()
    fields = pltpu.CompilerParams.__dataclass_fields__
    if "needs_layout_passes" in fields:
        cp = dataclasses.replace(cp, needs_layout_passes=False)
    if "use_tc_tiling_on_sc" in fields:
        cp = dataclasses.replace(cp, use_tc_tiling_on_sc=False)
    return cp


def _propagate(support, src3, dst3, w3):
    n_chunks = src3.shape[1]
    kfn = pl.kernel(
        _propagate_body(n_chunks),
        out_type=jax.ShapeDtypeStruct((_NC, _N_PAD, _HID), jnp.float32),
        mesh=plsc.VectorSubcoreMesh(
            core_axis_name="core", subcore_axis_name="subcore"),
        scratch_types=[
            pltpu.VMEM((n_chunks, _CHUNK), jnp.int32),
            pltpu.VMEM((n_chunks, _CHUNK), jnp.int32),
            pltpu.VMEM((n_chunks, _CHUNK), jnp.float32),
            pltpu.VMEM((_CHUNK, _HID), jnp.float32),
            pltpu.VMEM((_CHUNK, _HID), jnp.float32),
            pltpu.VMEM((_ROWS_PER_TILE, _HID), jnp.float32),
            pltpu.VMEM_SHARED((_N_PAD, _HID), jnp.float32),
            pltpu.SemaphoreType.DMA,
            pltpu.SemaphoreType.DMA,
            pltpu.SemaphoreType.DMA,
            pltpu.SemaphoreType.DMA,
        ],
        compiler_params=_sc_compiler_params(),
    )
    return kfn(support, src3, dst3, w3)[:, :_N_NODES]


def _mm1_body(x_ref, w_ref, o_ref):
    o_ref[...] = jnp.dot(x_ref[...], w_ref[...],
                         preferred_element_type=jnp.float32)


def _mm2_body(p_ref, b_ref, w_ref, o_ref):
    h = p_ref[0] + p_ref[1] + b_ref[...]
    h = jnp.maximum(h, 0.0)
    o_ref[...] = jnp.dot(h, w_ref[...], preferred_element_type=jnp.float32)


def _fin_body(q_ref, b_ref, o_ref):
    o_ref[...] = q_ref[0] + q_ref[1] + b_ref[...]


def kernel(feature, edge_index, edge_weight, W1, b1, W2, b2):
    n_edges = edge_index.shape[1]
    n_chunks = -(-n_edges // (_NW * _CHUNK))
    e_pad = _NW * n_chunks * _CHUNK

    src = edge_index[0].astype(jnp.int32)
    dst = edge_index[1].astype(jnp.int32)
    w = edge_weight.astype(jnp.float32)
    pad = e_pad - n_edges
    src3 = jnp.concatenate([src, jnp.zeros((pad,), jnp.int32)]
                           ).reshape(_NW, n_chunks, _CHUNK)
    dst3 = jnp.concatenate([dst, jnp.zeros((pad,), jnp.int32)]
                           ).reshape(_NW, n_chunks, _CHUNK)
    w3 = jnp.concatenate([w, jnp.zeros((pad,), jnp.float32)]
                         ).reshape(_NW, n_chunks, _CHUNK)

    n = feature.shape[0]
    support1 = pl.pallas_call(
        _mm1_body,
        out_shape=jax.ShapeDtypeStruct((n, _HID), jnp.float32),
    )(feature, W1)

    p = _propagate(support1, src3, dst3, w3)

    W2p = jnp.zeros((_HID, _HID), jnp.float32).at[:, :W2.shape[1]].set(W2)
    support2 = pl.pallas_call(
        _mm2_body,
        out_shape=jax.ShapeDtypeStruct((n, _HID), jnp.float32),
    )(p, b1.reshape(1, _HID), W2p)

    q = _propagate(support2, src3, dst3, w3)

    b2p = jnp.zeros((1, _HID), jnp.float32).at[0, :b2.shape[0]].set(b2)
    out16 = pl.pallas_call(
        _fin_body,
        out_shape=jax.ShapeDtypeStruct((n, _HID), jnp.float32),
    )(q, b2p)
    return out16[:, :b2.shape[0]]


# speedup vs baseline: 9.9750x; 1.0687x over previous
import dataclasses

import jax
import jax.numpy as jnp
from jax import lax
from jax.experimental import pallas as pl
from jax.experimental.pallas import tpu as pltpu
from jax.experimental.pallas import tpu_sc as plsc

_N_NODES = 10000
_HID = 16
_NC = 2
_NS = 16
_NW = _NC * _NS
_CHUNK = 128
_GATHER_DNUMS = lax.GatherDimensionNumbers(
    offset_dims=(), collapsed_slice_dims=(0,), start_index_map=(0,))


def _lane_broadcast(vec, lane):
    idx = jnp.full((_HID, 1), lane, jnp.int32)
    return lax.gather(vec, idx, _GATHER_DNUMS, slice_sizes=(1,),
                      mode=lax.GatherScatterMode.PROMISE_IN_BOUNDS)

_N_PAD = 10240
_ROWS_PER_TILE = _N_PAD // _NS


_NBUF = 4


def _propagate_body(n_chunks):
    assert n_chunks % _NBUF == 0 and n_chunks >= 2 * _NBUF

    def body(sup_hbm, src_hbm, dst_hbm, w_hbm, out_hbm,
             src_v, dst_v, w_v, rows0, rows1, rows2, rows3, zero_v, acc_sh,
             gsem0, gsem1, gsem2, gsem3, ssem0, ssem1, ssem2, ssem3):
        rows = (rows0, rows1, rows2, rows3)
        gsem = (gsem0, gsem1, gsem2, gsem3)
        ssem = (ssem0, ssem1, ssem2, ssem3)
        c = lax.axis_index("core")
        s = lax.axis_index("subcore")
        wid = c * _NS + s

        c_src = pltpu.async_copy(src_hbm.at[wid], src_v, gsem0)
        c_dst = pltpu.async_copy(dst_hbm.at[wid], dst_v, gsem1)
        c_w = pltpu.async_copy(w_hbm.at[wid], w_v, gsem2)
        c_src.wait()
        c_dst.wait()
        c_w.wait()

        for b in range(_NBUF):
            pltpu.async_copy(sup_hbm.at[src_v.at[b]], rows[b], gsem[b])

        @pl.loop(0, _ROWS_PER_TILE)
        def _zero(i):
            zero_v[i, :] = jnp.zeros((_HID,), jnp.float32)

        pltpu.sync_copy(
            zero_v, acc_sh.at[pl.ds(s * _ROWS_PER_TILE, _ROWS_PER_TILE)])
        plsc.subcore_barrier()

        def scale(rows, ch):
            @pl.loop(0, _CHUNK // _HID, unroll=4)
            def _grp(g):
                base = g * _HID
                wv16 = w_v[ch, pl.ds(base, _HID)]
                for j in range(_HID):
                    wj = _lane_broadcast(wv16, j)
                    rows[base + j, :] = rows[base + j, :] * wj

        @pl.loop(0, n_chunks, step=_NBUF)
        def _chunk(ch):
            scts = []
            for b in range(_NBUF):
                cur = ch + b
                pltpu.make_async_copy(
                    sup_hbm.at[src_v.at[cur]], rows[b], gsem[b]).wait()
                scale(rows[b], cur)
                scts.append(pltpu.async_copy(
                    rows[b], acc_sh.at[dst_v.at[cur]], ssem[b], add=True))

            for b in range(_NBUF):
                scts[b].wait()
                nxt = ch + _NBUF + b

                @pl.when(nxt < n_chunks)
                def _(b=b, nxt=nxt):
                    pltpu.async_copy(
                        sup_hbm.at[src_v.at[nxt]], rows[b], gsem[b])

        plsc.subcore_barrier()
        rs = pl.ds(s * _ROWS_PER_TILE, _ROWS_PER_TILE)
        pltpu.sync_copy(acc_sh.at[rs], out_hbm.at[c].at[rs])

    return body


def _sc_compiler_params():
    cp = pltpu.CompilerParams()
    fields = pltpu.CompilerParams.__dataclass_fields__
    if "needs_layout_passes" in fields:
        cp = dataclasses.replace(cp, needs_layout_passes=False)
    if "use_tc_tiling_on_sc" in fields:
        cp = dataclasses.replace(cp, use_tc_tiling_on_sc=False)
    return cp


def _propagate(support, src3, dst3, w3):
    n_chunks = src3.shape[1]
    kfn = pl.kernel(
        _propagate_body(n_chunks),
        out_type=jax.ShapeDtypeStruct((_NC, _N_PAD, _HID), jnp.float32),
        mesh=plsc.VectorSubcoreMesh(
            core_axis_name="core", subcore_axis_name="subcore"),
        scratch_types=[
            pltpu.VMEM((n_chunks, _CHUNK), jnp.int32),
            pltpu.VMEM((n_chunks, _CHUNK), jnp.int32),
            pltpu.VMEM((n_chunks, _CHUNK), jnp.float32),
            pltpu.VMEM((_CHUNK, _HID), jnp.float32),
            pltpu.VMEM((_CHUNK, _HID), jnp.float32),
            pltpu.VMEM((_CHUNK, _HID), jnp.float32),
            pltpu.VMEM((_CHUNK, _HID), jnp.float32),
            pltpu.VMEM((_ROWS_PER_TILE, _HID), jnp.float32),
            pltpu.VMEM_SHARED((_N_PAD, _HID), jnp.float32),
            pltpu.SemaphoreType.DMA,
            pltpu.SemaphoreType.DMA,
            pltpu.SemaphoreType.DMA,
            pltpu.SemaphoreType.DMA,
            pltpu.SemaphoreType.DMA,
            pltpu.SemaphoreType.DMA,
            pltpu.SemaphoreType.DMA,
            pltpu.SemaphoreType.DMA,
        ],
        compiler_params=_sc_compiler_params(),
    )
    return kfn(support, src3, dst3, w3)[:, :_N_NODES]


def _mm1_body(x_ref, w_ref, o_ref):
    o_ref[...] = jnp.dot(x_ref[...], w_ref[...],
                         preferred_element_type=jnp.float32)


def _mm2_body(p_ref, b_ref, w_ref, o_ref):
    h = p_ref[0] + p_ref[1] + b_ref[...]
    h = jnp.maximum(h, 0.0)
    o_ref[...] = jnp.dot(h, w_ref[...], preferred_element_type=jnp.float32)


def _fin_body(q_ref, b_ref, o_ref):
    o_ref[...] = q_ref[0] + q_ref[1] + b_ref[...]


def kernel(feature, edge_index, edge_weight, W1, b1, W2, b2):
    n_edges = edge_index.shape[1]
    n_chunks = -(-n_edges // (_NW * _CHUNK))
    e_pad = _NW * n_chunks * _CHUNK

    src = edge_index[0].astype(jnp.int32)
    dst = edge_index[1].astype(jnp.int32)
    w = edge_weight.astype(jnp.float32)
    pad = e_pad - n_edges
    src3 = jnp.concatenate([src, jnp.zeros((pad,), jnp.int32)]
                           ).reshape(_NW, n_chunks, _CHUNK)
    dst3 = jnp.concatenate([dst, jnp.zeros((pad,), jnp.int32)]
                           ).reshape(_NW, n_chunks, _CHUNK)
    w3 = jnp.concatenate([w, jnp.zeros((pad,), jnp.float32)]
                         ).reshape(_NW, n_chunks, _CHUNK)

    n = feature.shape[0]
    support1 = pl.pallas_call(
        _mm1_body,
        out_shape=jax.ShapeDtypeStruct((n, _HID), jnp.float32),
    )(feature, W1)

    p = _propagate(support1, src3, dst3, w3)

    W2p = jnp.zeros((_HID, _HID), jnp.float32).at[:, :W2.shape[1]].set(W2)
    support2 = pl.pallas_call(
        _mm2_body,
        out_shape=jax.ShapeDtypeStruct((n, _HID), jnp.float32),
    )(p, b1.reshape(1, _HID), W2p)

    q = _propagate(support2, src3, dst3, w3)

    b2p = jnp.zeros((1, _HID), jnp.float32).at[0, :b2.shape[0]].set(b2)
    out16 = pl.pallas_call(
        _fin_body,
        out_shape=jax.ShapeDtypeStruct((n, _HID), jnp.float32),
    )(q, b2p)
    return out16[:, :b2.shape[0]]


# speedup vs baseline: 11.0124x; 1.1040x over previous
import dataclasses

import jax
import jax.numpy as jnp
from jax import lax
from jax.experimental import pallas as pl
from jax.experimental.pallas import tpu as pltpu
from jax.experimental.pallas import tpu_sc as plsc

_N_NODES = 10000
_OUT = 7
_HID = 16
_NC = 2
_NS = 16
_NW = _NC * _NS
_CHUNK = 128
_GATHER_DNUMS = lax.GatherDimensionNumbers(
    offset_dims=(), collapsed_slice_dims=(0,), start_index_map=(0,))


def _lane_broadcast(vec, lane):
    idx = jnp.full((_HID, 1), lane, jnp.int32)
    return lax.gather(vec, idx, _GATHER_DNUMS, slice_sizes=(1,),
                      mode=lax.GatherScatterMode.PROMISE_IN_BOUNDS)

_N_PAD = 10240
_ROWS_PER_TILE = _N_PAD // _NS


_NBUF = 4


def _propagate_body(n_chunks):
    assert n_chunks % _NBUF == 0 and n_chunks >= 2 * _NBUF

    n_tile = n_chunks * _CHUNK

    def body(sup_hbm, src_hbm, dst_hbm, w_hbm, out_hbm,
             src_v, dst_v, w_v, rows0, rows1, rows2, rows3, zero_v, acc_sh,
             gsem0, gsem1, gsem2, gsem3, ssem0, ssem1, ssem2, ssem3):
        rows = (rows0, rows1, rows2, rows3)
        gsem = (gsem0, gsem1, gsem2, gsem3)
        ssem = (ssem0, ssem1, ssem2, ssem3)
        c = lax.axis_index("core")
        s = lax.axis_index("subcore")
        wid = c * _NS + s

        es = pl.ds(wid * n_tile, n_tile)
        c_src = pltpu.async_copy(src_hbm.at[es], src_v, gsem0)
        c_dst = pltpu.async_copy(dst_hbm.at[es], dst_v, gsem1)
        c_w = pltpu.async_copy(w_hbm.at[es], w_v, gsem2)
        c_src.wait()
        c_dst.wait()
        c_w.wait()

        def idx_at(ref, ch):
            return ref.at[pl.ds(ch * _CHUNK, _CHUNK)]

        for b in range(_NBUF):
            pltpu.async_copy(sup_hbm.at[idx_at(src_v, b)], rows[b], gsem[b])

        @pl.loop(0, _ROWS_PER_TILE)
        def _zero(i):
            zero_v[i, :] = jnp.zeros((_HID,), jnp.float32)

        pltpu.sync_copy(
            zero_v, acc_sh.at[pl.ds(s * _ROWS_PER_TILE, _ROWS_PER_TILE)])
        plsc.subcore_barrier()

        def scale(rows, ch):
            @pl.loop(0, _CHUNK // _HID, unroll=4)
            def _grp(g):
                base = g * _HID
                wv16 = w_v[pl.ds(ch * _CHUNK + base, _HID)]
                for j in range(_HID):
                    wj = _lane_broadcast(wv16, j)
                    rows[base + j, :] = rows[base + j, :] * wj

        @pl.loop(0, n_chunks, step=_NBUF)
        def _chunk(ch):
            scts = []
            for b in range(_NBUF):
                cur = ch + b
                pltpu.make_async_copy(
                    sup_hbm.at[idx_at(src_v, cur)], rows[b], gsem[b]).wait()
                scale(rows[b], cur)
                scts.append(pltpu.async_copy(
                    rows[b], acc_sh.at[idx_at(dst_v, cur)], ssem[b],
                    add=True))

            for b in range(_NBUF):
                scts[b].wait()
                nxt = ch + _NBUF + b

                @pl.when(nxt < n_chunks)
                def _(b=b, nxt=nxt):
                    pltpu.async_copy(
                        sup_hbm.at[idx_at(src_v, nxt)], rows[b], gsem[b])

        plsc.subcore_barrier()
        rs = pl.ds(s * _ROWS_PER_TILE, _ROWS_PER_TILE)
        pltpu.sync_copy(acc_sh.at[rs], out_hbm.at[c].at[rs])

    return body


def _sc_compiler_params():
    cp = pltpu.CompilerParams()
    fields = pltpu.CompilerParams.__dataclass_fields__
    if "needs_layout_passes" in fields:
        cp = dataclasses.replace(cp, needs_layout_passes=False)
    if "use_tc_tiling_on_sc" in fields:
        cp = dataclasses.replace(cp, use_tc_tiling_on_sc=False)
    return cp


def _propagate(support, src1, dst1, w1):
    n_chunks = src1.shape[0] // (_NW * _CHUNK)
    n_tile = n_chunks * _CHUNK
    kfn = pl.kernel(
        _propagate_body(n_chunks),
        out_type=jax.ShapeDtypeStruct((_NC, _N_PAD, _HID), jnp.float32),
        mesh=plsc.VectorSubcoreMesh(
            core_axis_name="core", subcore_axis_name="subcore"),
        scratch_types=[
            pltpu.VMEM((n_tile,), jnp.int32),
            pltpu.VMEM((n_tile,), jnp.int32),
            pltpu.VMEM((n_tile,), jnp.float32),
            pltpu.VMEM((_CHUNK, _HID), jnp.float32),
            pltpu.VMEM((_CHUNK, _HID), jnp.float32),
            pltpu.VMEM((_CHUNK, _HID), jnp.float32),
            pltpu.VMEM((_CHUNK, _HID), jnp.float32),
            pltpu.VMEM((_ROWS_PER_TILE, _HID), jnp.float32),
            pltpu.VMEM_SHARED((_N_PAD, _HID), jnp.float32),
            pltpu.SemaphoreType.DMA,
            pltpu.SemaphoreType.DMA,
            pltpu.SemaphoreType.DMA,
            pltpu.SemaphoreType.DMA,
            pltpu.SemaphoreType.DMA,
            pltpu.SemaphoreType.DMA,
            pltpu.SemaphoreType.DMA,
            pltpu.SemaphoreType.DMA,
        ],
        compiler_params=_sc_compiler_params(),
    )
    return kfn(support, src1, dst1, w1)


def _mm1_body(x_ref, w_ref, o_ref):
    o_ref[...] = jnp.dot(x_ref[...], w_ref[...],
                         preferred_element_type=jnp.float32)


def _mm2_body(p_ref, b_ref, w_ref, o_ref):
    h = p_ref[0] + p_ref[1] + b_ref[...]
    h = jnp.maximum(h, 0.0)
    o_ref[...] = jnp.dot(h, w_ref[...], preferred_element_type=jnp.float32)


def _fin_body(q_ref, b_ref, o_ref):
    tot = (q_ref[0, :_N_NODES, :] + q_ref[1, :_N_NODES, :]
           + b_ref[...])
    o_ref[...] = tot[:, :_OUT]


def kernel(feature, edge_index, edge_weight, W1, b1, W2, b2):
    n_edges = edge_index.shape[1]
    n_chunks = -(-n_edges // (_NW * _CHUNK))
    e_pad = _NW * n_chunks * _CHUNK

    src = edge_index[0].astype(jnp.int32)
    dst = edge_index[1].astype(jnp.int32)
    w = edge_weight.astype(jnp.float32)
    pad = e_pad - n_edges
    src1 = jnp.concatenate([src, jnp.zeros((pad,), jnp.int32)])
    dst1 = jnp.concatenate([dst, jnp.zeros((pad,), jnp.int32)])
    w1 = jnp.concatenate([w, jnp.zeros((pad,), jnp.float32)])

    n = feature.shape[0]
    support1 = pl.pallas_call(
        _mm1_body,
        out_shape=jax.ShapeDtypeStruct((n, _HID), jnp.float32),
    )(feature, W1)

    p = _propagate(support1, src1, dst1, w1)

    W2p = jnp.zeros((_HID, _HID), jnp.float32).at[:, :W2.shape[1]].set(W2)
    support2 = pl.pallas_call(
        _mm2_body,
        out_shape=jax.ShapeDtypeStruct((_N_PAD, _HID), jnp.float32),
    )(p, b1.reshape(1, _HID), W2p)

    q = _propagate(support2, src1, dst1, w1)

    b2p = jnp.zeros((1, _HID), jnp.float32).at[0, :b2.shape[0]].set(b2)
    return pl.pallas_call(
        _fin_body,
        out_shape=jax.ShapeDtypeStruct((_N_NODES, _OUT), jnp.float32),
    )(q, b2p)
